# baseline (speedup 1.0000x reference)
import functools

import jax
import jax.numpy as jnp
from jax import lax
from jax.experimental import pallas as pl
from jax.experimental.pallas import tpu as pltpu
from jax.experimental.pallas import tpu_sc as plsc

N = 10000
E = 320000
NCORES = 2
NSUB = 16
F2 = 128
NPAD = 10240
RPT = NPAD // NSUB
K = 88
EPAD = 321024
EPT_SPLIT = EPAD // NSUB
EPT_PART = EPAD // (2 * NSUB)


def _make_sc_agg(split_features):
    ept = EPT_SPLIT if split_features else EPT_PART
    nch = ept // K
    NBR = 4
    NBI = 8
    row_shape, dtype = (F2,), jnp.float32
    scratch = [
        pltpu.VMEM((NBI, K), jnp.int32),
        pltpu.VMEM((NBI, K), jnp.int32),
        pltpu.VMEM((NBR, K) + row_shape, dtype),
        pltpu.VMEM_SHARED((NPAD,) + row_shape, dtype),
    ] + [pltpu.SemaphoreType.DMA] * (2 * NBR + 2 * NBI)
    mesh = plsc.VectorSubcoreMesh(core_axis_name="c", subcore_axis_name="s")

    def body(x_hbm, src_hbm, dst_hbm, zer_hbm, agg_hbm, idxs, idxd, rows,
             acc, *sems):
        gsem = sems[:NBR]
        ssem = sems[NBR:2 * NBR]
        is_sem = sems[2 * NBR:2 * NBR + NBI]
        id_sem = sems[2 * NBR + NBI:]
        c = lax.axis_index("c")
        s = lax.axis_index("s")
        r0 = s * RPT
        if split_features:
            cbase = s * ept
        else:
            cbase = c * (EPAD // 2) + s * ept

        def start_is(j, slot):
            pltpu.async_copy(src_hbm.at[pl.ds(cbase + j * K, K)],
                             idxs.at[slot], is_sem[slot])

        def start_id(j, slot):
            pltpu.async_copy(dst_hbm.at[pl.ds(cbase + j * K, K)],
                             idxd.at[slot], id_sem[slot])

        def wait_is(slot):
            pltpu.make_async_copy(src_hbm.at[pl.ds(cbase, K)],
                                  idxs.at[slot], is_sem[slot]).wait()

        def wait_id(slot):
            pltpu.make_async_copy(dst_hbm.at[pl.ds(cbase, K)],
                                  idxd.at[slot], id_sem[slot]).wait()

        def xsrc(islot):
            tab = x_hbm.at[c] if split_features else x_hbm
            return tab.at[idxs.at[islot]]

        def start_g(islot, rslot):
            pltpu.async_copy(xsrc(islot), rows.at[rslot], gsem[rslot])

        def wait_g(islot, rslot):
            pltpu.make_async_copy(xsrc(islot), rows.at[rslot],
                                  gsem[rslot]).wait()

        def start_s(islot, rslot):
            pltpu.async_copy(rows.at[rslot], acc.at[idxd.at[islot]],
                             ssem[rslot], add=True)

        def wait_s(islot, rslot):
            pltpu.make_async_copy(rows.at[rslot], acc.at[idxd.at[islot]],
                                  ssem[rslot]).wait()

        pltpu.sync_copy(zer_hbm.at[pl.ds(r0, RPT)], acc.at[pl.ds(r0, RPT)])
        for b in range(NBI):
            start_is(b, b)
            start_id(b, b)
        for b in range(NBR - 1):
            wait_is(b)
            wait_id(b)
            start_g(b, b)
        plsc.subcore_barrier()

        def step(i, carry):
            for b in range(NBI):
                j = i * NBI + b
                rb = b % NBR
                rbp = (b - 1) % NBR
                ibp = (b - 1) % NBI
                ibg = (b - 1 + NBR) % NBI

                @pl.when(j < nch)
                def _():
                    wait_g(b, rb)
                    start_s(b, rb)

                @pl.when(j + NBI < nch)
                def _():
                    start_is(j + NBI, b)

                @pl.when((j >= 1) & (j - 1 < nch))
                def _():
                    wait_s(ibp, rbp)

                @pl.when((j >= 1) & (j - 1 + NBI < nch))
                def _():
                    start_id(j - 1 + NBI, ibp)

                @pl.when(j - 1 + NBR < nch)
                def _():
                    wait_is(ibg)
                    wait_id(ibg)
                    start_g(ibg, rbp)
            return carry

        lax.fori_loop(0, (nch + NBI - 1) // NBI, step, 0)
        if nch % NBI == 0:
            wait_s((nch - 1) % NBI, (nch - 1) % NBR)
        plsc.subcore_barrier()
        pltpu.sync_copy(acc.at[pl.ds(r0, RPT)], agg_hbm.at[c].at[pl.ds(r0, RPT)])

    return pl.kernel(
        body,
        out_type=jax.ShapeDtypeStruct((NCORES, NPAD) + row_shape, dtype),
        mesh=mesh, scratch_types=scratch)


def _make_sc_cnt():
    scratch = [
        pltpu.VMEM((EPT_PART,), jnp.int32),
        pltpu.VMEM((NPAD,), jnp.float32),
        pltpu.VMEM((NSUB, RPT), jnp.float32),
        pltpu.VMEM((RPT,), jnp.float32),
        pltpu.VMEM_SHARED((NSUB, NPAD), jnp.float32),
    ]
    mesh = plsc.VectorSubcoreMesh(core_axis_name="c", subcore_axis_name="s")

    def body(dst_hbm, cnt_hbm, idxd, hist, cols, red, stage):
        ones16 = jnp.ones((16,), jnp.float32)
        c = lax.axis_index("c")
        s = lax.axis_index("s")
        r0 = s * RPT
        base = c * (EPAD // 2) + s * EPT_PART
        pltpu.sync_copy(dst_hbm.at[pl.ds(base, EPT_PART)], idxd)

        def zero(i, carry):
            hist[pl.ds(i * 16, 16)] = jnp.zeros((16,), jnp.float32)
            return carry
        lax.fori_loop(0, NPAD // 16, zero, 0)

        def chunk(g, carry):
            ii = idxd[pl.ds(g * 16, 16)]
            plsc.addupdate_scatter(hist, [ii], ones16)
            return carry
        lax.fori_loop(0, EPT_PART // 16, chunk, 0)

        pltpu.sync_copy(hist, stage.at[s])
        plsc.subcore_barrier()
        pltpu.sync_copy(stage.at[:, pl.ds(r0, RPT)], cols)

        def tree(j, carry):
            acc16 = cols[0, pl.ds(j * 16, 16)]
            for i in range(1, NSUB):
                acc16 = acc16 + cols[i, pl.ds(j * 16, 16)]
            red[pl.ds(j * 16, 16)] = acc16
            return carry
        lax.fori_loop(0, RPT // 16, tree, 0)
        pltpu.sync_copy(red, cnt_hbm.at[c].at[pl.ds(r0, RPT)])

    return pl.kernel(
        body,
        out_type=jax.ShapeDtypeStruct((NCORES, NPAD), jnp.float32),
        mesh=mesh, scratch_types=scratch,
        compiler_params=pltpu.CompilerParams(needs_layout_passes=False))


def _make_tc_layer(split_in, Fo, relu, split_out, R=2000):
    F2o = Fo // 2
    grid = (N // R,)
    in_specs = [
        pl.BlockSpec((NCORES, R, F2), lambda i: (0, i, 0)),
        (pl.BlockSpec((NCORES, R, F2), lambda i: (0, i, 0)) if split_in
         else pl.BlockSpec((R, F2), lambda i: (i, 0))),
        pl.BlockSpec((NCORES, R, 1), lambda i: (0, i, 0)),
        pl.BlockSpec(((4 if split_in else 2) * F2, Fo), lambda i: (0, 0)),
        pl.BlockSpec((1, Fo), lambda i: (0, 0)),
    ]
    if split_out:
        out_specs = pl.BlockSpec((NCORES, R, F2o), lambda i: (0, i, 0))
        out_shape = jax.ShapeDtypeStruct((NCORES, N, F2o), jnp.float32)
    else:
        out_specs = pl.BlockSpec((R, Fo), lambda i: (i, 0))
        out_shape = jax.ShapeDtypeStruct((N, Fo), jnp.float32)

    def body(agg_ref, x_ref, cnt_ref, w_ref, b_ref, o_ref):
        inv = 1.0 / jnp.maximum(cnt_ref[0] + cnt_ref[1], 1.0)
        if split_in:
            a = jnp.concatenate([agg_ref[0], agg_ref[1]], axis=1) * inv
            xx = jnp.concatenate([x_ref[0], x_ref[1]], axis=1)
        else:
            a = (agg_ref[0] + agg_ref[1]) * inv
            xx = x_ref[...]
        h = jnp.dot(jnp.concatenate([a, xx], axis=1), w_ref[...],
                    preferred_element_type=jnp.float32) + b_ref[...]
        if relu:
            h = jnp.maximum(h, 0.0)
        if split_out:
            o_ref[0] = h[:, :F2o]
            o_ref[1] = h[:, F2o:]
        else:
            o_ref[...] = h

    return pl.pallas_call(body, grid=grid, in_specs=in_specs,
                          out_specs=out_specs, out_shape=out_shape)


def _make_tc_layer2(R=2000):
    grid = (N // R,)
    in_specs = [
        pl.BlockSpec((NCORES, R, F2), lambda i: (0, i, 0)),
        pl.BlockSpec((NCORES, R, F2), lambda i: (0, i, 0)),
        pl.BlockSpec((NCORES, R, 1), lambda i: (0, i, 0)),
        pl.BlockSpec((4 * F2, 256), lambda i: (0, 0)),
        pl.BlockSpec((1, 256), lambda i: (0, 0)),
        pl.BlockSpec((256, F2), lambda i: (0, 0)),
        pl.BlockSpec((256, F2), lambda i: (0, 0)),
    ]
    out_specs = [
        pl.BlockSpec((R, F2), lambda i: (i, 0)),
        pl.BlockSpec((R, F2), lambda i: (i, 0)),
    ]
    out_shape = [jax.ShapeDtypeStruct((N, F2), jnp.float32)] * 2

    def body(agg_ref, x_ref, cnt_ref, w_ref, b_ref, wl3_ref, wr3_ref,
             y_ref, z_ref):
        inv = 1.0 / jnp.maximum(cnt_ref[0] + cnt_ref[1], 1.0)
        a = jnp.concatenate([agg_ref[0], agg_ref[1]], axis=1) * inv
        xx = jnp.concatenate([x_ref[0], x_ref[1]], axis=1)
        h = jnp.dot(jnp.concatenate([a, xx], axis=1), w_ref[...],
                    preferred_element_type=jnp.float32) + b_ref[...]
        h = jnp.maximum(h, 0.0)
        y_ref[...] = jnp.dot(h, wl3_ref[...],
                             preferred_element_type=jnp.float32)
        z_ref[...] = jnp.dot(h, wr3_ref[...],
                             preferred_element_type=jnp.float32)

    return pl.pallas_call(body, grid=grid, in_specs=in_specs,
                          out_specs=out_specs, out_shape=out_shape)


def _make_tc_layer3(R=2000):
    grid = (N // R,)
    in_specs = [
        pl.BlockSpec((NCORES, R, F2), lambda i: (0, i, 0)),
        pl.BlockSpec((R, F2), lambda i: (i, 0)),
        pl.BlockSpec((NCORES, R, 1), lambda i: (0, i, 0)),
        pl.BlockSpec((1, F2), lambda i: (0, 0)),
    ]
    out_specs = pl.BlockSpec((R, F2), lambda i: (i, 0))
    out_shape = jax.ShapeDtypeStruct((N, F2), jnp.float32)

    def body(agg_ref, z_ref, cnt_ref, b_ref, o_ref):
        inv = 1.0 / jnp.maximum(cnt_ref[0] + cnt_ref[1], 1.0)
        mean = (agg_ref[0] + agg_ref[1]) * inv
        o_ref[...] = jnp.maximum(mean + z_ref[...] + b_ref[...], 0.0)

    return pl.pallas_call(body, grid=grid, in_specs=in_specs,
                          out_specs=out_specs, out_shape=out_shape)


@functools.cache
def _pipeline():
    sc_part = _make_sc_agg(split_features=False)
    sc_wide = _make_sc_agg(split_features=True)
    sc_cnt = _make_sc_cnt()
    tc1 = _make_tc_layer(split_in=False, Fo=256, relu=True, split_out=True)
    tc2 = _make_tc_layer2()
    tc3 = _make_tc_layer3()
    tc4 = _make_tc_layer(split_in=False, Fo=128, relu=False, split_out=False)
    return sc_part, sc_wide, sc_cnt, tc1, tc2, tc3, tc4


def kernel(z, edge_index, Wl1, Wr1, b1, Wl2, Wr2, b2, Wl3, Wr3, b3,
           Wl4, Wr4, b4):
    sc_part, sc_wide, sc_cnt, tc1, tc2, tc3, tc4 = _pipeline()
    pad = EPAD - E
    src = jnp.concatenate([edge_index[0], jnp.zeros((pad,), jnp.int32)])
    dst = jnp.concatenate([edge_index[1],
                           jnp.full((pad,), NPAD - 1, jnp.int32)])
    zer = jnp.zeros((NPAD, F2), jnp.float32)

    def wcat(Wl, Wr):
        return jnp.concatenate([Wl.T, Wr.T], axis=0)

    cnt = sc_cnt(dst).reshape(NCORES, NPAD, 1)
    agg1 = sc_part(z, src, dst, zer)
    x2 = tc1(agg1, z, cnt, wcat(Wl1, Wr1), b1.reshape(1, -1))
    agg2 = sc_wide(x2, src, dst, zer)
    y3, z3 = tc2(agg2, x2, cnt, wcat(Wl2, Wr2), b2.reshape(1, -1),
                 Wl3.T, Wr3.T)
    agg3 = sc_part(y3, src, dst, zer)
    x4 = tc3(agg3, z3, cnt, b3.reshape(1, -1))
    agg4 = sc_part(x4, src, dst, zer)
    return tc4(agg4, x4, cnt, wcat(Wl4, Wr4), b4.reshape(1, -1))


# speedup vs baseline: 1.2828x; 1.2828x over previous
import functools

import jax
import jax.numpy as jnp
from jax import lax
from jax.experimental import pallas as pl
from jax.experimental.pallas import tpu as pltpu
from jax.experimental.pallas import tpu_sc as plsc

N = 10000
E = 320000
NCORES = 2
NSUB = 16
F2 = 128
NPAD = 10240
RPT = NPAD // NSUB
K = 80
EPAD = E
EPT_SPLIT = EPAD // NSUB
EPT_PART = EPAD // (2 * NSUB)


def _make_sc_agg(split_features):
    ept = EPT_SPLIT if split_features else EPT_PART
    nch = ept // K
    NBR = 4
    NBI = 8
    row_shape, dtype = (F2,), jnp.float32
    scratch = [
        pltpu.VMEM((NBI, K), jnp.int32),
        pltpu.VMEM((NBI, K), jnp.int32),
        pltpu.VMEM((NBR, K) + row_shape, dtype),
        pltpu.VMEM_SHARED((NPAD,) + row_shape, dtype),
    ] + [pltpu.SemaphoreType.DMA] * (2 * NBR + 2 * NBI)
    mesh = plsc.VectorSubcoreMesh(core_axis_name="c", subcore_axis_name="s")

    def body(x_hbm, src_hbm, dst_hbm, zer_hbm, agg_hbm, idxs, idxd, rows,
             acc, *sems):
        gsem = sems[:NBR]
        ssem = sems[NBR:2 * NBR]
        is_sem = sems[2 * NBR:2 * NBR + NBI]
        id_sem = sems[2 * NBR + NBI:]
        c = lax.axis_index("c")
        s = lax.axis_index("s")
        r0 = s * RPT
        if split_features:
            cbase = s * ept
        else:
            cbase = c * (EPAD // 2) + s * ept

        def start_is(j, slot):
            pltpu.async_copy(src_hbm.at[pl.ds(cbase + j * K, K)],
                             idxs.at[slot], is_sem[slot])

        def start_id(j, slot):
            pltpu.async_copy(dst_hbm.at[pl.ds(cbase + j * K, K)],
                             idxd.at[slot], id_sem[slot])

        def wait_is(slot):
            pltpu.make_async_copy(src_hbm.at[pl.ds(cbase, K)],
                                  idxs.at[slot], is_sem[slot]).wait()

        def wait_id(slot):
            pltpu.make_async_copy(dst_hbm.at[pl.ds(cbase, K)],
                                  idxd.at[slot], id_sem[slot]).wait()

        def xsrc(islot):
            tab = x_hbm.at[c] if split_features else x_hbm
            return tab.at[idxs.at[islot]]

        def start_g(islot, rslot):
            pltpu.async_copy(xsrc(islot), rows.at[rslot], gsem[rslot])

        def wait_g(islot, rslot):
            pltpu.make_async_copy(xsrc(islot), rows.at[rslot],
                                  gsem[rslot]).wait()

        def start_s(islot, rslot):
            pltpu.async_copy(rows.at[rslot], acc.at[idxd.at[islot]],
                             ssem[rslot], add=True)

        def wait_s(islot, rslot):
            pltpu.make_async_copy(rows.at[rslot], acc.at[idxd.at[islot]],
                                  ssem[rslot]).wait()

        pltpu.sync_copy(zer_hbm.at[pl.ds(r0, RPT)], acc.at[pl.ds(r0, RPT)])
        for b in range(NBI):
            start_is(b, b)
            start_id(b, b)
        for b in range(NBR - 1):
            wait_is(b)
            wait_id(b)
            start_g(b, b)
        plsc.subcore_barrier()

        def step(i, carry):
            for b in range(NBI):
                j = i * NBI + b
                rb = b % NBR
                rbp = (b - 1) % NBR
                ibp = (b - 1) % NBI
                ibg = (b - 1 + NBR) % NBI

                @pl.when(j < nch)
                def _():
                    wait_g(b, rb)
                    start_s(b, rb)

                @pl.when(j + NBI < nch)
                def _():
                    start_is(j + NBI, b)

                @pl.when((j >= 1) & (j - 1 < nch))
                def _():
                    wait_s(ibp, rbp)

                @pl.when((j >= 1) & (j - 1 + NBI < nch))
                def _():
                    start_id(j - 1 + NBI, ibp)

                @pl.when(j - 1 + NBR < nch)
                def _():
                    wait_is(ibg)
                    wait_id(ibg)
                    start_g(ibg, rbp)
            return carry

        lax.fori_loop(0, (nch + NBI - 1) // NBI, step, 0)
        if nch % NBI == 0:
            wait_s((nch - 1) % NBI, (nch - 1) % NBR)
        plsc.subcore_barrier()
        pltpu.sync_copy(acc.at[pl.ds(r0, RPT)], agg_hbm.at[c].at[pl.ds(r0, RPT)])

    return pl.kernel(
        body,
        out_type=jax.ShapeDtypeStruct((NCORES, NPAD) + row_shape, dtype),
        mesh=mesh, scratch_types=scratch)


def _make_sc_cnt():
    scratch = [
        pltpu.VMEM((EPT_PART,), jnp.int32),
        pltpu.VMEM((NPAD,), jnp.float32),
        pltpu.VMEM((NSUB, RPT), jnp.float32),
        pltpu.VMEM((RPT,), jnp.float32),
        pltpu.VMEM_SHARED((NSUB, NPAD), jnp.float32),
    ]
    mesh = plsc.VectorSubcoreMesh(core_axis_name="c", subcore_axis_name="s")

    def body(dst_hbm, cnt_hbm, idxd, hist, cols, red, stage):
        ones16 = jnp.ones((16,), jnp.float32)
        c = lax.axis_index("c")
        s = lax.axis_index("s")
        r0 = s * RPT
        base = c * (EPAD // 2) + s * EPT_PART
        pltpu.sync_copy(dst_hbm.at[pl.ds(base, EPT_PART)], idxd)

        def zero(i, carry):
            hist[pl.ds(i * 16, 16)] = jnp.zeros((16,), jnp.float32)
            return carry
        lax.fori_loop(0, NPAD // 16, zero, 0)

        def chunk(g, carry):
            ii = idxd[pl.ds(g * 16, 16)]
            plsc.addupdate_scatter(hist, [ii], ones16)
            return carry
        lax.fori_loop(0, EPT_PART // 16, chunk, 0)

        pltpu.sync_copy(hist, stage.at[s])
        plsc.subcore_barrier()
        pltpu.sync_copy(stage.at[:, pl.ds(r0, RPT)], cols)

        def tree(j, carry):
            acc16 = cols[0, pl.ds(j * 16, 16)]
            for i in range(1, NSUB):
                acc16 = acc16 + cols[i, pl.ds(j * 16, 16)]
            red[pl.ds(j * 16, 16)] = acc16
            return carry
        lax.fori_loop(0, RPT // 16, tree, 0)
        pltpu.sync_copy(red, cnt_hbm.at[c].at[pl.ds(r0, RPT)])

    return pl.kernel(
        body,
        out_type=jax.ShapeDtypeStruct((NCORES, NPAD), jnp.float32),
        mesh=mesh, scratch_types=scratch,
        compiler_params=pltpu.CompilerParams(needs_layout_passes=False))


def _make_tc_layer(split_in, Fo, relu, split_out, R=2000):
    F2o = Fo // 2
    grid = (N // R,)
    in_specs = [
        pl.BlockSpec((NCORES, R, F2), lambda i: (0, i, 0)),
        (pl.BlockSpec((NCORES, R, F2), lambda i: (0, i, 0)) if split_in
         else pl.BlockSpec((R, F2), lambda i: (i, 0))),
        pl.BlockSpec((NCORES, R, 1), lambda i: (0, i, 0)),
        pl.BlockSpec(((4 if split_in else 2) * F2, Fo), lambda i: (0, 0)),
        pl.BlockSpec((1, Fo), lambda i: (0, 0)),
    ]
    if split_out:
        out_specs = pl.BlockSpec((NCORES, R, F2o), lambda i: (0, i, 0))
        out_shape = jax.ShapeDtypeStruct((NCORES, N, F2o), jnp.float32)
    else:
        out_specs = pl.BlockSpec((R, Fo), lambda i: (i, 0))
        out_shape = jax.ShapeDtypeStruct((N, Fo), jnp.float32)

    def body(agg_ref, x_ref, cnt_ref, w_ref, b_ref, o_ref):
        inv = 1.0 / jnp.maximum(cnt_ref[0] + cnt_ref[1], 1.0)
        if split_in:
            a = jnp.concatenate([agg_ref[0], agg_ref[1]], axis=1) * inv
            xx = jnp.concatenate([x_ref[0], x_ref[1]], axis=1)
        else:
            a = (agg_ref[0] + agg_ref[1]) * inv
            xx = x_ref[...]
        h = jnp.dot(jnp.concatenate([a, xx], axis=1), w_ref[...],
                    preferred_element_type=jnp.float32) + b_ref[...]
        if relu:
            h = jnp.maximum(h, 0.0)
        if split_out:
            o_ref[0] = h[:, :F2o]
            o_ref[1] = h[:, F2o:]
        else:
            o_ref[...] = h

    return pl.pallas_call(body, grid=grid, in_specs=in_specs,
                          out_specs=out_specs, out_shape=out_shape)


def _make_tc_layer2(R=2000):
    grid = (N // R,)
    in_specs = [
        pl.BlockSpec((NCORES, R, F2), lambda i: (0, i, 0)),
        pl.BlockSpec((NCORES, R, F2), lambda i: (0, i, 0)),
        pl.BlockSpec((NCORES, R, 1), lambda i: (0, i, 0)),
        pl.BlockSpec((4 * F2, 256), lambda i: (0, 0)),
        pl.BlockSpec((1, 256), lambda i: (0, 0)),
        pl.BlockSpec((256, F2), lambda i: (0, 0)),
        pl.BlockSpec((256, F2), lambda i: (0, 0)),
    ]
    out_specs = [
        pl.BlockSpec((R, F2), lambda i: (i, 0)),
        pl.BlockSpec((R, F2), lambda i: (i, 0)),
    ]
    out_shape = [jax.ShapeDtypeStruct((N, F2), jnp.float32)] * 2

    def body(agg_ref, x_ref, cnt_ref, w_ref, b_ref, wl3_ref, wr3_ref,
             y_ref, z_ref):
        inv = 1.0 / jnp.maximum(cnt_ref[0] + cnt_ref[1], 1.0)
        a = jnp.concatenate([agg_ref[0], agg_ref[1]], axis=1) * inv
        xx = jnp.concatenate([x_ref[0], x_ref[1]], axis=1)
        h = jnp.dot(jnp.concatenate([a, xx], axis=1), w_ref[...],
                    preferred_element_type=jnp.float32) + b_ref[...]
        h = jnp.maximum(h, 0.0)
        y_ref[...] = jnp.dot(h, wl3_ref[...],
                             preferred_element_type=jnp.float32)
        z_ref[...] = jnp.dot(h, wr3_ref[...],
                             preferred_element_type=jnp.float32)

    return pl.pallas_call(body, grid=grid, in_specs=in_specs,
                          out_specs=out_specs, out_shape=out_shape)


def _make_tc_layer3(R=2000):
    grid = (N // R,)
    in_specs = [
        pl.BlockSpec((NCORES, R, F2), lambda i: (0, i, 0)),
        pl.BlockSpec((R, F2), lambda i: (i, 0)),
        pl.BlockSpec((NCORES, R, 1), lambda i: (0, i, 0)),
        pl.BlockSpec((1, F2), lambda i: (0, 0)),
    ]
    out_specs = pl.BlockSpec((R, F2), lambda i: (i, 0))
    out_shape = jax.ShapeDtypeStruct((N, F2), jnp.float32)

    def body(agg_ref, z_ref, cnt_ref, b_ref, o_ref):
        inv = 1.0 / jnp.maximum(cnt_ref[0] + cnt_ref[1], 1.0)
        mean = (agg_ref[0] + agg_ref[1]) * inv
        o_ref[...] = jnp.maximum(mean + z_ref[...] + b_ref[...], 0.0)

    return pl.pallas_call(body, grid=grid, in_specs=in_specs,
                          out_specs=out_specs, out_shape=out_shape)


@functools.cache
def _pipeline():
    sc_part = _make_sc_agg(split_features=False)
    sc_wide = _make_sc_agg(split_features=True)
    sc_cnt = _make_sc_cnt()
    tc1 = _make_tc_layer(split_in=False, Fo=256, relu=True, split_out=True)
    tc2 = _make_tc_layer2()
    tc3 = _make_tc_layer3()
    tc4 = _make_tc_layer(split_in=False, Fo=128, relu=False, split_out=False)
    return sc_part, sc_wide, sc_cnt, tc1, tc2, tc3, tc4


def kernel(z, edge_index, Wl1, Wr1, b1, Wl2, Wr2, b2, Wl3, Wr3, b3,
           Wl4, Wr4, b4):
    sc_part, sc_wide, sc_cnt, tc1, tc2, tc3, tc4 = _pipeline()
    src = edge_index[0]
    dst = edge_index[1]
    zer = jnp.zeros((NPAD, F2), jnp.float32)

    def wcat(Wl, Wr):
        return jnp.concatenate([Wl.T, Wr.T], axis=0)

    cnt = sc_cnt(dst).reshape(NCORES, NPAD, 1)
    agg1 = sc_part(z, src, dst, zer)
    x2 = tc1(agg1, z, cnt, wcat(Wl1, Wr1), b1.reshape(1, -1))
    agg2 = sc_wide(x2, src, dst, zer)
    y3, z3 = tc2(agg2, x2, cnt, wcat(Wl2, Wr2), b2.reshape(1, -1),
                 Wl3.T, Wr3.T)
    agg3 = sc_part(y3, src, dst, zer)
    x4 = tc3(agg3, z3, cnt, b3.reshape(1, -1))
    agg4 = sc_part(x4, src, dst, zer)
    return tc4(agg4, x4, cnt, wcat(Wl4, Wr4), b4.reshape(1, -1))
